# baseline (speedup 1.0000x reference)
import functools

import jax
import jax.numpy as jnp
from jax import lax
from jax.experimental import pallas as pl
from jax.experimental.pallas import tpu as pltpu
from jax.experimental.pallas import tpu_sc as plsc

_NC = 2
_NS = 16
_NW = _NC * _NS
_BPG = 8


@functools.lru_cache(maxsize=None)
def _make_gather(bsz, hist, D):
    bpw = bsz // _NW
    n_groups = bpw // _BPG
    grp_rows = _BPG * hist
    mesh = plsc.VectorSubcoreMesh(core_axis_name="c", subcore_axis_name="s")

    @functools.partial(
        pl.kernel,
        mesh=mesh,
        out_type=jax.ShapeDtypeStruct((bsz, hist, D), jnp.float32),
        compiler_params=pltpu.CompilerParams(use_tc_tiling_on_sc=False),
        scratch_types=[
            pltpu.VMEM((bpw * hist,), jnp.int32),
            pltpu.VMEM((2, grp_rows, D), jnp.float32),
            pltpu.SemaphoreType.DMA,
            pltpu.SemaphoreType.DMA,
            pltpu.SemaphoreType.DMA,
            pltpu.SemaphoreType.DMA,
        ],
    )
    def _kern(idx_hbm, table_hbm, out_hbm, idx_v, rows_v, gs0, gs1, ws0, ws1):
        gs = [gs0, gs1]
        ws = [ws0, ws1]
        wid = lax.axis_index("s") * _NC + lax.axis_index("c")
        base_b = wid * bpw
        pltpu.sync_copy(idx_hbm.at[wid], idx_v)

        def gather_grp(g, s):
            return pltpu.make_async_copy(
                table_hbm.at[idx_v.at[pl.ds(g * grp_rows, grp_rows)]],
                rows_v.at[s],
                gs[s],
            )

        def write_grp(g, s):
            for b in range(_BPG):
                yield pltpu.make_async_copy(
                    rows_v.at[s].at[pl.ds(b * hist, hist)],
                    out_hbm.at[base_b + g * _BPG + b],
                    ws[s],
                )

        gather_grp(0, 0).start()

        def body(i, carry):
            for s in range(2):
                g = 2 * i + s
                gather_grp(g, s).wait()
                for c in write_grp(g, s):
                    c.start()

                @pl.when(g >= 1)
                def _():
                    for c in write_grp(g - 1, 1 - s):
                        c.wait()

                @pl.when(g + 1 < n_groups)
                def _():
                    gather_grp(g + 1, 1 - s).start()

            return carry

        lax.fori_loop(0, n_groups // 2, body, 0)
        for c in write_grp(n_groups - 1, 1):
            c.wait()

    return _kern


def kernel(x, weight):
    bsz, hist = x.shape
    n_emb, dim = weight.shape
    idx = x.reshape(_NW, (bsz // _NW) * hist).astype(jnp.int32)
    return _make_gather(bsz, hist, dim)(idx, weight)


# speedup vs baseline: 1.0052x; 1.0052x over previous
import functools

import jax
import jax.numpy as jnp
from jax import lax
from jax.experimental import pallas as pl
from jax.experimental.pallas import tpu as pltpu
from jax.experimental.pallas import tpu_sc as plsc

_NC = 2
_NS = 16
_NW = _NC * _NS
_CHUNK = 128
_GPC = 5


@functools.lru_cache(maxsize=None)
def _make_gather(B, D):
    n_chunks = B // _CHUNK
    cpw = n_chunks // _NW
    mesh = plsc.VectorSubcoreMesh(core_axis_name="c", subcore_axis_name="s")

    @functools.partial(
        pl.kernel,
        mesh=mesh,
        out_type=jax.ShapeDtypeStruct((B, D), jnp.float32),
        compiler_params=pltpu.CompilerParams(use_tc_tiling_on_sc=False),
        scratch_types=[
            pltpu.VMEM((cpw * _CHUNK,), jnp.int32),
            pltpu.VMEM((2, _GPC * _CHUNK, D), jnp.float32),
            pltpu.SemaphoreType.DMA,
            pltpu.SemaphoreType.DMA,
            pltpu.SemaphoreType.DMA,
            pltpu.SemaphoreType.DMA,
        ],
    )
    def _kern(idx_hbm, table_hbm, out_hbm, idx_v, rows_v, gs0, gs1, ws0, ws1):
        gs = [gs0, gs1]
        ws = [ws0, ws1]
        n_groups = cpw // _GPC
        grp_rows = _GPC * _CHUNK
        wid = lax.axis_index("s") * _NC + lax.axis_index("c")
        base_row = wid * cpw * _CHUNK
        pltpu.sync_copy(idx_hbm.at[wid], idx_v)

        def gather_grp(g, s):
            return pltpu.make_async_copy(
                table_hbm.at[idx_v.at[pl.ds(g * grp_rows, grp_rows)]],
                rows_v.at[s],
                gs[s],
            )

        def write_grp(g, s):
            return pltpu.make_async_copy(
                rows_v.at[s],
                out_hbm.at[pl.ds(base_row + g * grp_rows, grp_rows)],
                ws[s],
            )

        gather_grp(0, 0).start()

        def body(i, carry):
            for s in range(2):
                g = 2 * i + s
                gather_grp(g, s).wait()
                write_grp(g, s).start()

                @pl.when(g >= 1)
                def _():
                    write_grp(g - 1, 1 - s).wait()

                @pl.when(g + 1 < n_groups)
                def _():
                    gather_grp(g + 1, 1 - s).start()

            return carry

        lax.fori_loop(0, n_groups // 2, body, 0)
        write_grp(n_groups - 1, 1).wait()

    return _kern


def kernel(x, weight):
    bsz, hist = x.shape
    n_emb, dim = weight.shape
    B = bsz * hist
    idx = x.reshape(_NW, B // _NW).astype(jnp.int32)
    out = _make_gather(B, dim)(idx, weight)
    return out.reshape(bsz, hist, dim)
